# baseline (speedup 1.0000x reference)
import functools
import math

import jax
import jax.numpy as jnp
from jax import lax
from jax.experimental import pallas as pl
from jax.experimental.pallas import tpu as pltpu
from jax.experimental.pallas import tpu_sc as plsc

_LANES = 16
_CHUNK = 128


def _make_sc_gather(n_idx_rows: int, v: int, d: int, scale: float,
                    num_cores: int, num_subcores: int):
    nw = num_cores * num_subcores
    rows_per_w = n_idx_rows // nw
    n_per_w = rows_per_w * _CHUNK
    mesh = plsc.VectorSubcoreMesh(core_axis_name="c", subcore_axis_name="s")

    @functools.partial(
        pl.kernel,
        out_type=jax.ShapeDtypeStruct((n_idx_rows * _CHUNK, d), jnp.float32),
        mesh=mesh,
        scratch_types=[
            pltpu.VMEM((rows_per_w, _CHUNK), jnp.int32),
            pltpu.VMEM((_CHUNK, d), jnp.float32),
            pltpu.SemaphoreType.DMA,
        ],
        compiler_params=pltpu.CompilerParams(use_tc_tiling_on_sc=False),
    )
    def sc_gather(idx_hbm, table_hbm, out_hbm, idx_v, rows_v, sem):
        wid = lax.axis_index("s") * num_cores + lax.axis_index("c")
        base = wid * n_per_w
        pltpu.sync_copy(idx_hbm.at[pl.ds(wid * rows_per_w, rows_per_w)], idx_v)

        def chunk_body(j, _):
            pltpu.async_copy(table_hbm.at[idx_v.at[j]], rows_v, sem).wait()

            def scale_body(r, _):
                for col in range(d // _LANES):
                    sl = pl.ds(col * _LANES, _LANES)
                    rows_v[r, sl] = rows_v[r, sl] * scale
                return 0

            lax.fori_loop(0, _CHUNK, scale_body, 0)
            pltpu.sync_copy(rows_v, out_hbm.at[pl.ds(base + j * _CHUNK, _CHUNK)])
            return 0

        lax.fori_loop(0, rows_per_w, chunk_body, 0)

    return sc_gather


def kernel(tokens, table):
    b, s = tokens.shape
    v, d = table.shape
    n = b * s
    info = plsc.get_sparse_core_info()
    nw = info.num_cores * info.num_subcores
    assert n % (nw * _CHUNK) == 0 and d % _LANES == 0
    idx = tokens.reshape(n // _CHUNK, _CHUNK).astype(jnp.int32)
    out = _make_sc_gather(n // _CHUNK, v, d, math.sqrt(d), info.num_cores,
                          info.num_subcores)(idx, table)
    return out.reshape(b, s, d)


# speedup vs baseline: 1.2047x; 1.2047x over previous
import functools
import math

import jax
import jax.numpy as jnp
from jax import lax
from jax.experimental import pallas as pl
from jax.experimental.pallas import tpu as pltpu
from jax.experimental.pallas import tpu_sc as plsc

_LANES = 16
_CHUNK = 128
_CPB = 2
_NBUF = 4


def _make_sc_gather(n_idx_rows: int, v: int, d: int, scale: float,
                    num_cores: int, num_subcores: int):
    nw = num_cores * num_subcores
    rows_per_w = n_idx_rows // nw
    n_per_w = rows_per_w * _CHUNK
    iters = rows_per_w // _CPB
    groups = iters // _NBUF
    rpb = _CPB * _CHUNK
    assert iters % _NBUF == 0 and iters >= 2 * _NBUF
    mesh = plsc.VectorSubcoreMesh(core_axis_name="c", subcore_axis_name="s")

    @functools.partial(
        pl.kernel,
        out_type=jax.ShapeDtypeStruct((n_idx_rows * _CHUNK, d), jnp.float32),
        mesh=mesh,
        scratch_types=[
            pltpu.VMEM((rows_per_w, _CHUNK), jnp.int32),
            pltpu.VMEM((_NBUF, rpb, d), jnp.float32),
            pltpu.SemaphoreType.DMA((_NBUF,)),
            pltpu.SemaphoreType.DMA((_NBUF,)),
        ],
        compiler_params=pltpu.CompilerParams(use_tc_tiling_on_sc=False),
    )
    def sc_gather(idx_hbm, table_hbm, out_hbm, idx_v, rows_v, gsem, ssem):
        wid = lax.axis_index("s") * num_cores + lax.axis_index("c")
        base = wid * n_per_w
        pltpu.sync_copy(idx_hbm.at[pl.ds(wid * rows_per_w, rows_per_w)], idx_v)

        def issue_gathers(it, b):
            for k in range(_CPB):
                pltpu.async_copy(
                    table_hbm.at[idx_v.at[it * _CPB + k]],
                    rows_v.at[b, pl.ds(k * _CHUNK, _CHUNK)],
                    gsem.at[b],
                )

        def wait_gathers(b):
            pltpu.make_async_copy(
                out_hbm.at[pl.ds(base, rpb)], rows_v.at[b], gsem.at[b]
            ).wait()

        def wait_store(b):
            pltpu.make_async_copy(
                rows_v.at[b], out_hbm.at[pl.ds(base, rpb)], ssem.at[b]
            ).wait()

        for b in range(_NBUF):
            issue_gathers(b, b)

        def group_body(g, carry):
            i0 = g * _NBUF
            for b in range(_NBUF):
                i = i0 + b
                bprev = (b - 1) % _NBUF

                @pl.when(jnp.logical_and(i >= 1, i + _NBUF - 1 < iters))
                def _():
                    wait_store(bprev)
                    issue_gathers(i + _NBUF - 1, bprev)

                wait_gathers(b)

                @plsc.parallel_loop(0, rpb, unroll=4)
                def _(r):
                    for col in range(d // _LANES):
                        sl = pl.ds(col * _LANES, _LANES)
                        rows_v[b, r, sl] = rows_v[b, r, sl] * scale

                pltpu.async_copy(
                    rows_v.at[b],
                    out_hbm.at[pl.ds(base + i * rpb, rpb)],
                    ssem.at[b],
                )
            return carry

        lax.fori_loop(0, groups, group_body, 0)
        for b in range(_NBUF):
            wait_store(b)

    return sc_gather


def kernel(tokens, table):
    b, s = tokens.shape
    v, d = table.shape
    n = b * s
    info = plsc.get_sparse_core_info()
    nw = info.num_cores * info.num_subcores
    assert n % (nw * _CHUNK * _CPB * _NBUF) == 0 and d % _LANES == 0
    idx = tokens.reshape(n // _CHUNK, _CHUNK).astype(jnp.int32)
    out = _make_sc_gather(n // _CHUNK, v, d, math.sqrt(d), info.num_cores,
                          info.num_subcores)(idx, table)
    return out.reshape(b, s, d)
